# baseline (speedup 1.0000x reference)
import jax
import jax.numpy as jnp
from jax.experimental import pallas as pl
from jax.experimental.pallas import tpu as pltpu

_N0 = 8
_NM = 7
_NB = 16
_C = 128
_PIX = 256


def _conv_band_matrix(conv_w):
    eyes = jnp.stack([jnp.eye(16, k=3 - k, dtype=jnp.float32)
                      for k in range(7)])
    mats = []
    for d in range(2):
        m4 = jnp.einsum('kab,kl,lcd->acbd', eyes, conv_w[0, d], eyes,
                        precision=jax.lax.Precision.HIGHEST)
        mats.append(m4.reshape(256, 256))
    return jnp.concatenate(mats, axis=0)


def _agg_kernel(x4_ref, m_ref, b_ref, out_ref, t_ref):
    t_ref[...] = x4_ref[...]
    bias = b_ref[0, 0]
    mband = m_ref[...]

    row8 = jax.lax.broadcasted_iota(jnp.int32, (_NB, _N0, _N0), 1)
    col8 = jax.lax.broadcasted_iota(jnp.int32, (_NB, _N0, _N0), 2)
    flat8 = row8 * _N0 + col8
    lane8 = jax.lax.broadcasted_iota(jnp.int32, (_NB, 1, _N0), 2)
    inf = jnp.float32(jnp.inf)
    bigi = jnp.int32(2**30)

    x4 = x4_ref[...]
    d_rows, sq_cols = [], []
    for m in range(_N0):
        gm = jnp.sum(x4[:, m:] * x4[:, m:m + 1], axis=(2, 3))
        sq_cols.append(gm[:, 0:1])
        d_rows.append(gm)
    sq3 = jnp.concatenate(sq_cols, axis=1)[:, None, :]
    rows16 = []
    for m in range(_N0):
        r = sq_cols[m] + jnp.concatenate(sq_cols[m:], axis=1) - 2.0 * d_rows[m]
        if m:
            r = jnp.concatenate(
                [jnp.full((_NB, m), inf), r], axis=1)
        rows16.append(r[:, None, :])
    D = jnp.concatenate(rows16, axis=1)

    act_r = jnp.ones((_NB, _N0, _N0), jnp.float32)
    act_c = jnp.ones((_NB, _N0, _N0), jnp.float32)
    refidx = lane8

    v4 = None
    for k in range(_NM):
        valid = (act_r > 0.5) & (act_c > 0.5) & (row8 != col8)
        deff = jnp.where(valid, D, inf)
        dmin = jnp.min(deff, axis=(1, 2), keepdims=True)
        fidx = jnp.min(jnp.where(deff == dmin, flat8, bigi),
                       axis=(1, 2), keepdims=True)
        sa = fidx // _N0
        sb = fidx - sa * _N0
        ra = jnp.sum(jnp.where(lane8 == sa, refidx, 0), axis=2, keepdims=True)
        rb = jnp.sum(jnp.where(lane8 == sb, refidx, 0), axis=2, keepdims=True)
        swap = ra > rb
        sl = jnp.where(swap, sb, sa)
        sr = jnp.where(swap, sa, sb)

        pairs = []
        for b in range(_NB):
            xl = t_ref[b, sl[b, 0, 0]]
            xr = t_ref[b, sr[b, 0, 0]]
            pairs.append(jnp.concatenate([xl, xr], axis=1))
        pair_all = jnp.concatenate(pairs, axis=0)
        v = jax.lax.dot_general(
            pair_all, mband, (((1,), (0,)), ((), ()))) + bias
        v4 = v.reshape(_NB, _C, _PIX)
        for b in range(_NB):
            t_ref[b, sl[b, 0, 0]] = v4[b]

        g3 = jnp.sum(t_ref[...] * v4[:, None], axis=(2, 3))[:, None, :]
        sqn = jnp.sum(jnp.where(lane8 == sl, g3, 0.0),
                      axis=2, keepdims=True)
        dnew = sq3 + sqn - 2.0 * g3
        sq3 = jnp.where(lane8 == sl, sqn, sq3)
        D = jnp.where(col8 == sl, inf, D)
        D = jnp.where(row8 == sl, dnew, D)
        act_r = jnp.where(row8 == sr, 0.0, act_r)
        act_c = jnp.where(col8 == sr, 0.0, act_c)
        refidx = jnp.where(lane8 == sl, _N0 + k, refidx)

    out_ref[...] = v4


def kernel(x, conv_w, conv_b):
    b, n0, c, w, h = x.shape
    pix = w * h
    xr = x.reshape(b, n0, c, pix)
    mband = _conv_band_matrix(conv_w)
    bias = conv_b.reshape(1, 1)
    out = pl.pallas_call(
        _agg_kernel,
        grid=(b // _NB,),
        in_specs=[
            pl.BlockSpec((_NB, n0, c, pix), lambda i: (i, 0, 0, 0)),
            pl.BlockSpec((2 * pix, pix), lambda i: (0, 0)),
            pl.BlockSpec((1, 1), lambda i: (0, 0)),
        ],
        out_specs=pl.BlockSpec((_NB, c, pix), lambda i: (i, 0, 0)),
        out_shape=jax.ShapeDtypeStruct((b, c, pix), jnp.float32),
        scratch_shapes=[pltpu.VMEM((_NB, _N0, c, pix), jnp.float32)],
        compiler_params=pltpu.CompilerParams(
            dimension_semantics=("arbitrary",)),
    )(xr, mband, bias)
    return out.reshape(b, c, w, h)


# speedup vs baseline: 1.0083x; 1.0083x over previous
import jax
import jax.numpy as jnp
from jax.experimental import pallas as pl
from jax.experimental.pallas import tpu as pltpu

_N0 = 8
_NM = 7
_NS = 16
_NB = 16
_C = 128
_PIX = 256


def _conv_band_matrix(conv_w):
    eyes = jnp.stack([jnp.eye(16, k=3 - k, dtype=jnp.float32)
                      for k in range(7)])
    mats = []
    for d in range(2):
        m4 = jnp.einsum('kab,kl,lcd->acbd', eyes, conv_w[0, d], eyes,
                        precision=jax.lax.Precision.HIGHEST)
        mats.append(m4.reshape(256, 256))
    return jnp.concatenate(mats, axis=0)


def _agg_kernel(x4_ref, m_ref, b_ref, out_ref, t_ref):
    t_ref[:, 0:_N0] = x4_ref[...]
    bias = b_ref[0, 0]
    mband = m_ref[...]

    row3 = jax.lax.broadcasted_iota(jnp.int32, (_NB, _NS, _NS), 1)
    col3 = jax.lax.broadcasted_iota(jnp.int32, (_NB, _NS, _NS), 2)
    flat3 = row3 * _NS + col3
    ci16 = jax.lax.broadcasted_iota(jnp.int32, (1, _NS), 1)
    inf = jnp.float32(jnp.inf)
    bigi = jnp.int32(2**30)

    x4 = x4_ref[...]
    grows = []
    for m in range(_N0):
        gm = jnp.sum(x4 * x4[:, m:m + 1], axis=(2, 3))
        grows.append(gm[:, None, :])
    G = jnp.concatenate(grows, axis=1)
    r8 = jax.lax.broadcasted_iota(jnp.int32, (_NB, _N0, _N0), 1)
    c8 = jax.lax.broadcasted_iota(jnp.int32, (_NB, _N0, _N0), 2)
    gdiag = jnp.where(r8 == c8, G, 0.0)
    sqr = jnp.sum(gdiag, axis=1, keepdims=True)
    sqc = jnp.sum(gdiag, axis=2, keepdims=True)
    db = sqc + sqr - 2.0 * G
    db = jnp.concatenate(
        [db, jnp.full((_NB, _N0, _NS - _N0), inf)], axis=2)
    D = jnp.concatenate(
        [db, jnp.full((_NB, _NS - _N0, _NS), inf)], axis=1)
    sq = jnp.concatenate(
        [sqr[:, 0, :], jnp.zeros((_NB, _NS - _N0), jnp.float32)], axis=1)

    act_r = (row3 < _N0).astype(jnp.float32)
    act_c = (col3 < _N0).astype(jnp.float32)

    v4 = None
    for k in range(_NM):
        p = _N0 + k
        valid = (act_r > 0.5) & (act_c > 0.5) & (row3 > col3)
        deff = jnp.where(valid, D, inf)
        dmin = jnp.min(deff, axis=(1, 2), keepdims=True)
        fidx = jnp.min(jnp.where(deff == dmin, flat3, bigi),
                       axis=(1, 2), keepdims=True)
        jv = fidx // _NS
        iv = fidx - jv * _NS

        pairs = []
        for b in range(_NB):
            xl = t_ref[b, iv[b, 0, 0]]
            xr = t_ref[b, jv[b, 0, 0]]
            pairs.append(jnp.concatenate([xl, xr], axis=1))
        pair_all = jnp.concatenate(pairs, axis=0)
        v = jax.lax.dot_general(
            pair_all, mband, (((1,), (0,)), ((), ()))) + bias
        v4 = v.reshape(_NB, _C, _PIX)
        for b in range(_NB):
            t_ref[b, p] = v4[b]

        g = jnp.sum(t_ref[:, 0:p + 1] * v4[:, None], axis=(2, 3))
        sq_p = g[:, p:p + 1]
        g16 = jnp.concatenate(
            [g, jnp.zeros((_NB, _NS - p - 1), jnp.float32)], axis=1)
        dnew = sq + sq_p - 2.0 * g16
        D = jnp.where(row3 == p, dnew[:, None, :], D)
        sq = jnp.where(ci16 == p, sq_p, sq)

        act_r = jnp.where((row3 == iv) | (row3 == jv), 0.0, act_r)
        act_c = jnp.where((col3 == iv) | (col3 == jv), 0.0, act_c)
        act_r = jnp.where(row3 == p, 1.0, act_r)
        act_c = jnp.where(col3 == p, 1.0, act_c)

    out_ref[...] = v4


def kernel(x, conv_w, conv_b):
    b, n0, c, w, h = x.shape
    pix = w * h
    xr = x.reshape(b, n0, c, pix)
    mband = _conv_band_matrix(conv_w)
    bias = conv_b.reshape(1, 1)
    grid = b // _NB
    out = pl.pallas_call(
        _agg_kernel,
        grid=(grid,),
        in_specs=[
            pl.BlockSpec((_NB, n0, c, pix), lambda i: (i, 0, 0, 0)),
            pl.BlockSpec((2 * pix, pix), lambda i: (0, 0)),
            pl.BlockSpec((1, 1), lambda i: (0, 0)),
        ],
        out_specs=pl.BlockSpec((_NB, c, pix), lambda i: (i, 0, 0)),
        out_shape=jax.ShapeDtypeStruct((b, c, pix), jnp.float32),
        scratch_shapes=[pltpu.VMEM((_NB, _NS, c, pix), jnp.float32)],
        compiler_params=pltpu.CompilerParams(
            dimension_semantics=("arbitrary",)),
    )(xr, mband, bias)
    return out.reshape(b, c, w, h)
